# baseline (speedup 1.0000x reference)
import functools

import jax
import jax.numpy as jnp
from jax.experimental import pallas as pl
from jax.experimental.pallas import tpu as pltpu
from jax.experimental.pallas import tpu_sc as plsc

EMB = 256
NH = 4
DH = 64
NCLS = 8
NROW = NCLS * NH
NEG = -1e30


def _dot(a, b, ca, cb):
    return jax.lax.dot_general(
        a, b, (((ca,), (cb,)), ((), ())), preferred_element_type=jnp.float32)


def _acc_kernel(nt, x_ref, cls_ref, sem_ref, Wi_ref, bi_ref, Ws_ref, bs_ref,
                Wo_ref, bo_ref, recW_ref, recb_ref, gateW_ref, gateb_ref,
                ng_ref, nb_ref, fused_ref,
                qt_ref, m_ref, l_ref, Y_ref, Z_ref, cnt_ref):
    i = pl.program_id(0)
    T = x_ref.shape[0]
    f32 = jnp.float32

    @pl.when(i == 0)
    def _init():
        sem = sem_ref[...]
        Wq = Wi_ref[0:EMB, :]
        Wk = Wi_ref[EMB:2 * EMB, :]
        bq = bi_ref[0:1, :]
        Q = _dot(sem, Wq, 1, 1) + bq
        rr = jax.lax.broadcasted_iota(jnp.int32, (NROW, NCLS), 0) // NH
        sel = (rr == jax.lax.broadcasted_iota(jnp.int32, (NROW, NCLS), 1)).astype(f32)
        Qexp = _dot(sel, Q, 1, 0)
        hh = jax.lax.broadcasted_iota(jnp.int32, (NROW, EMB), 0) % NH
        ee = jax.lax.broadcasted_iota(jnp.int32, (NROW, EMB), 1) // DH
        Qmask = Qexp * (hh == ee).astype(f32)
        t = _dot(Qmask, Wk, 1, 0)
        qt = _dot(t, Ws_ref[...], 1, 0)
        qt_ref[...] = qt * (1.0 / 8.0)
        m_ref[...] = jnp.full((NROW, 128), NEG, f32)
        l_ref[...] = jnp.zeros((NROW, 128), f32)
        Y_ref[...] = jnp.zeros((NROW, EMB), f32)
        Z_ref[...] = jnp.zeros((NCLS, EMB), f32)
        cnt_ref[...] = jnp.zeros((NCLS, 128), f32)

    xt = x_ref[...]
    clsrow = cls_ref[0]
    ST = _dot(qt_ref[...], xt, 1, 1)
    ccls = jax.lax.broadcasted_iota(jnp.int32, (NROW, T), 0) // NH
    msk = ccls == clsrow
    STm = jnp.where(msk, ST, NEG)
    tmax = jnp.max(STm, axis=1, keepdims=True)
    mold = m_ref[:, 0:1]
    mnew = jnp.maximum(mold, tmax)
    resc = jnp.exp(mold - mnew)
    P = jnp.where(msk, jnp.exp(STm - mnew), 0.0)
    l_ref[...] = l_ref[...] * resc + jnp.sum(P, axis=1, keepdims=True)
    Y_ref[...] = Y_ref[...] * resc + _dot(P, xt, 1, 0)
    m_ref[...] = jnp.broadcast_to(mnew, (NROW, 128))

    c8 = jax.lax.broadcasted_iota(jnp.int32, (NCLS, T), 0)
    P8 = (c8 == clsrow).astype(f32)
    cnt_ref[...] += jnp.sum(P8, axis=1, keepdims=True)
    Z_ref[...] += _dot(P8, xt, 1, 0)

    @pl.when(i == nt - 1)
    def _fin():
        Ws = Ws_ref[...]
        bs = bs_ref[...]
        ybar = Y_ref[...] / l_ref[:, 0:1]
        U = _dot(ybar, Ws, 1, 1) + bs
        Wv = Wi_ref[2 * EMB:3 * EMB, :]
        bv = bi_ref[2:3, :]
        Vf = _dot(U, Wv, 1, 1) + bv
        hh2 = jax.lax.broadcasted_iota(jnp.int32, (NROW, EMB), 0) % NH
        ee2 = jax.lax.broadcasted_iota(jnp.int32, (NROW, EMB), 1) // DH
        Vm = Vf * (hh2 == ee2).astype(jnp.float32)
        rr2 = jax.lax.broadcasted_iota(jnp.int32, (NCLS, NROW), 1) // NH
        sel2 = (rr2 == jax.lax.broadcasted_iota(jnp.int32, (NCLS, NROW), 0)).astype(jnp.float32)
        attheads = _dot(sel2, Vm, 1, 0)
        att = _dot(attheads, Wo_ref[...], 1, 1) + bo_ref[...]
        old = _dot(Z_ref[...] / cnt_ref[:, 0:1], Ws, 1, 1) + bs
        sem = sem_ref[...]
        recW = recW_ref[...]
        new = (_dot(sem, recW[:, 0:EMB], 1, 1)
               + _dot(att, recW[:, EMB:2 * EMB], 1, 1) + recb_ref[...])
        gW = gateW_ref[...]
        g = jax.nn.sigmoid(_dot(old, gW[:, 0:EMB], 1, 1)
                           + _dot(new, gW[:, EMB:2 * EMB], 1, 1) + gateb_ref[...])
        fused = g * old + (1.0 - g) * new
        mu = jnp.mean(fused, axis=1, keepdims=True)
        var = jnp.mean((fused - mu) ** 2, axis=1, keepdims=True)
        fused_ref[...] = ((fused - mu) / jnp.sqrt(var + 1e-5) * ng_ref[...]
                          + nb_ref[...])


def _sc_apply(x, cls, fused):
    N = x.shape[0]
    NW = 32
    per_w = N // NW
    CH = 64
    nch = per_w // CH
    NBUF = 2
    mesh = plsc.VectorSubcoreMesh(core_axis_name="c", subcore_axis_name="s")

    @functools.partial(
        pl.kernel, mesh=mesh,
        out_type=jax.ShapeDtypeStruct((N, EMB), jnp.float32),
        scratch_types=[
            pltpu.VMEM((per_w,), jnp.int32),
            pltpu.VMEM((NBUF, CH, EMB), jnp.float32),
            pltpu.VMEM((NBUF, CH, EMB), jnp.float32),
            pltpu.SemaphoreType.DMA,
            pltpu.SemaphoreType.DMA,
            pltpu.SemaphoreType.DMA,
        ],
    )
    def body(x_hbm, cls_hbm, fused_hbm, out_hbm, idx_v, xb, rb,
             semx, semg, semo):
        wid = jax.lax.axis_index("s") * 2 + jax.lax.axis_index("c")
        base = wid * per_w
        pltpu.sync_copy(cls_hbm.at[pl.ds(base, per_w)], idx_v)

        def start_x(ci):
            b = ci % NBUF
            xc = pltpu.async_copy(
                x_hbm.at[pl.ds(base + ci * CH, CH), :], xb.at[b], semx)
            gc = pltpu.async_copy(
                fused_hbm.at[idx_v.at[pl.ds(ci * CH, CH)]], rb.at[b], semg)
            return xc, gc

        cps = start_x(0)
        ocp_prev = None
        for ci in range(nch):
            b = ci % NBUF
            cps_cur = cps
            if ci + 1 < nch:
                if ocp_prev is not None:
                    ocp_prev.wait()
                cps = start_x(ci + 1)
            cps_cur[0].wait()
            cps_cur[1].wait()

            def row(r, carry, b=b):
                for g in range(EMB // 16):
                    sl = pl.ds(g * 16, 16)
                    rb[b, r, sl] = rb[b, r, sl] * xb[b, r, sl]
                return carry

            jax.lax.fori_loop(0, CH, row, 0)
            ocp_prev = pltpu.async_copy(
                rb.at[b], out_hbm.at[pl.ds(base + ci * CH, CH), :], semo)
        ocp_prev.wait()

    return body(x, cls.astype(jnp.int32), fused)


def _apply_kernel(x_ref, cls_ref, fused_ref, out_ref):
    T = x_ref.shape[0]
    clsrow = cls_ref[0]
    c8 = jax.lax.broadcasted_iota(jnp.int32, (NCLS, T), 0)
    P8 = (c8 == clsrow).astype(jnp.float32)
    g = _dot(P8, fused_ref[...], 0, 0)
    out_ref[...] = g * x_ref[...]


def _side(x, sem, Wi, bi, Ws, bs, Wo, bo, recW, recb, gateW, gateb, ng, nb,
          cls, T):
    N = x.shape[0]
    nt = N // T
    cls3 = cls.astype(jnp.int32).reshape(nt, 1, T)
    bi3 = bi.reshape(3, EMB)
    row = lambda a: a.reshape(1, EMB)
    full = lambda s: pl.BlockSpec(s, lambda i: (0,) * len(s))

    fused = pl.pallas_call(
        functools.partial(_acc_kernel, nt),
        grid=(nt,),
        in_specs=[
            pl.BlockSpec((T, EMB), lambda i: (i, 0)),
            pl.BlockSpec((1, 1, T), lambda i: (i, 0, 0)),
            full((NCLS, EMB)), full((3 * EMB, EMB)), full((3, EMB)),
            full((EMB, EMB)), full((1, EMB)),
            full((EMB, EMB)), full((1, EMB)),
            full((EMB, 2 * EMB)), full((1, EMB)),
            full((EMB, 2 * EMB)), full((1, EMB)),
            full((1, EMB)), full((1, EMB)),
        ],
        out_specs=full((NCLS, EMB)),
        out_shape=jax.ShapeDtypeStruct((NCLS, EMB), jnp.float32),
        scratch_shapes=[
            pltpu.VMEM((NROW, EMB), jnp.float32),
            pltpu.VMEM((NROW, 128), jnp.float32),
            pltpu.VMEM((NROW, 128), jnp.float32),
            pltpu.VMEM((NROW, EMB), jnp.float32),
            pltpu.VMEM((NCLS, EMB), jnp.float32),
            pltpu.VMEM((NCLS, 128), jnp.float32),
        ],
    )(x, cls3, sem, Wi, bi3, Ws, row(bs), Wo, row(bo), recW, row(recb),
      gateW, row(gateb), row(ng), row(nb))

    return _sc_apply(x, cls, fused)


def kernel(v, c, v_sem, c_sem, params, v_class, c_class):
    p = params
    v_upd = _side(v, v_sem, p['av_Wi'], p['av_bi'], p['send_var_W'],
                  p['send_var_b'], p['av_Wo'], p['av_bo'], p['rec_var_W'],
                  p['rec_var_b'], p['gate_v_W'], p['gate_v_b'], p['norm_g'],
                  p['norm_b'], v_class, 2048)
    c_upd = _side(c, c_sem, p['ac_Wi'], p['ac_bi'], p['send_con_W'],
                  p['send_con_b'], p['ac_Wo'], p['ac_bo'], p['rec_con_W'],
                  p['rec_con_b'], p['gate_c_W'], p['gate_c_b'], p['norm_g'],
                  p['norm_b'], c_class, 2048)
    return v_upd, c_upd


# speedup vs baseline: 4.5210x; 4.5210x over previous
import functools

import jax
import jax.numpy as jnp
from jax.experimental import pallas as pl
from jax.experimental.pallas import tpu as pltpu

EMB = 256
NH = 4
DH = 64
NCLS = 8
NROW = NCLS * NH
NACC = NROW + NCLS
NEG = -1e30


def _dot(a, b, ca, cb):
    return jax.lax.dot_general(
        a, b, (((ca,), (cb,)), ((), ())), preferred_element_type=jnp.float32)


def _side_kernel(nt, x_ref, cls_ref, sem_ref, Wi_ref, bi_ref, Ws_ref, bs_ref,
                 Wo_ref, bo_ref, recW_ref, recb_ref, gateW_ref, gateb_ref,
                 ng_ref, nb_ref, out_ref,
                 qt_ref, m_ref, l_ref, Y_ref, fused_ref):
    i = pl.program_id(0)
    T = x_ref.shape[0]
    f32 = jnp.float32

    @pl.when(i == 0)
    def _init():
        sem = sem_ref[...]
        Wq = Wi_ref[0:EMB, :]
        Wk = Wi_ref[EMB:2 * EMB, :]
        bq = bi_ref[0:1, :]
        Q = _dot(sem, Wq, 1, 1) + bq
        rr = jax.lax.broadcasted_iota(jnp.int32, (NROW, NCLS), 0) // NH
        sel = (rr == jax.lax.broadcasted_iota(jnp.int32, (NROW, NCLS), 1)).astype(f32)
        Qexp = _dot(sel, Q, 1, 0)
        hh = jax.lax.broadcasted_iota(jnp.int32, (NROW, EMB), 0) % NH
        ee = jax.lax.broadcasted_iota(jnp.int32, (NROW, EMB), 1) // DH
        Qmask = Qexp * (hh == ee).astype(f32)
        t = _dot(Qmask, Wk, 1, 0)
        qt = _dot(t, Ws_ref[...], 1, 0)
        qt_ref[...] = qt * (1.0 / 8.0)
        m_ref[...] = jnp.full((NROW, 128), NEG, f32)
        l_ref[...] = jnp.zeros((NACC, 128), f32)
        Y_ref[...] = jnp.zeros((NACC, EMB), f32)

    @pl.when(i < nt)
    def _acc():
        xt = x_ref[...]
        clsrow = cls_ref[0]
        ST = _dot(qt_ref[...], xt, 1, 1)
        ccls = jax.lax.broadcasted_iota(jnp.int32, (NROW, T), 0) // NH
        msk = ccls == clsrow
        STm = jnp.where(msk, ST, NEG)
        tmax = jnp.max(STm, axis=1, keepdims=True)
        mold = m_ref[:, 0:1]
        mnew = jnp.maximum(mold, tmax)
        resc = jnp.exp(mold - mnew)
        P = jnp.where(msk, jnp.exp(STm - mnew), 0.0)
        c8 = jax.lax.broadcasted_iota(jnp.int32, (NCLS, T), 0)
        P8 = (c8 == clsrow).astype(f32)
        P40 = jnp.concatenate((P, P8), axis=0)
        resc40 = jnp.concatenate((resc, jnp.ones((NCLS, 1), f32)), axis=0)
        l_ref[...] = l_ref[...] * resc40 + jnp.sum(P40, axis=1, keepdims=True)
        Y_ref[...] = Y_ref[...] * resc40 + _dot(P40, xt, 1, 0)
        m_ref[...] = jnp.broadcast_to(mnew, (NROW, 128))

    @pl.when(i == nt - 1)
    def _fin():
        Ws = Ws_ref[...]
        bs = bs_ref[...]
        ybar = Y_ref[0:NROW, :] / l_ref[0:NROW, 0:1]
        U = _dot(ybar, Ws, 1, 1) + bs
        Wv = Wi_ref[2 * EMB:3 * EMB, :]
        bv = bi_ref[2:3, :]
        Vf = _dot(U, Wv, 1, 1) + bv
        hh2 = jax.lax.broadcasted_iota(jnp.int32, (NROW, EMB), 0) % NH
        ee2 = jax.lax.broadcasted_iota(jnp.int32, (NROW, EMB), 1) // DH
        Vm = Vf * (hh2 == ee2).astype(jnp.float32)
        rr2 = jax.lax.broadcasted_iota(jnp.int32, (NCLS, NROW), 1) // NH
        sel2 = (rr2 == jax.lax.broadcasted_iota(jnp.int32, (NCLS, NROW), 0)).astype(jnp.float32)
        attheads = _dot(sel2, Vm, 1, 0)
        att = _dot(attheads, Wo_ref[...], 1, 1) + bo_ref[...]
        old = _dot(Y_ref[NROW:NACC, :] / l_ref[NROW:NACC, 0:1], Ws, 1, 1) + bs
        sem = sem_ref[...]
        recW = recW_ref[...]
        new = (_dot(sem, recW[:, 0:EMB], 1, 1)
               + _dot(att, recW[:, EMB:2 * EMB], 1, 1) + recb_ref[...])
        gW = gateW_ref[...]
        g = jax.nn.sigmoid(_dot(old, gW[:, 0:EMB], 1, 1)
                           + _dot(new, gW[:, EMB:2 * EMB], 1, 1) + gateb_ref[...])
        fused = g * old + (1.0 - g) * new
        mu = jnp.mean(fused, axis=1, keepdims=True)
        var = jnp.mean((fused - mu) ** 2, axis=1, keepdims=True)
        fused_ref[...] = ((fused - mu) / jnp.sqrt(var + 1e-5) * ng_ref[...]
                          + nb_ref[...])

    @pl.when(i >= nt)
    def _apply():
        xt = x_ref[...]
        clsrow = cls_ref[0]
        c8 = jax.lax.broadcasted_iota(jnp.int32, (NCLS, T), 0)
        P8 = (c8 == clsrow).astype(jnp.float32)
        g = _dot(P8, fused_ref[...], 0, 0)
        out_ref[...] = g * xt


def _side(x, sem, Wi, bi, Ws, bs, Wo, bo, recW, recb, gateW, gateb, ng, nb,
          cls, T):
    N = x.shape[0]
    nt = N // T
    cls3 = cls.astype(jnp.int32).reshape(nt, 1, T)
    bi3 = bi.reshape(3, EMB)
    row = lambda a: a.reshape(1, EMB)
    full = lambda s: pl.BlockSpec(s, lambda i: (0,) * len(s))

    def tile_map(i):
        return (jnp.where(i < nt, i, i - nt), 0)

    def tile_map3(i):
        return (jnp.where(i < nt, i, i - nt), 0, 0)

    def out_map(i):
        return (jnp.maximum(i - nt, 0), 0)

    out = pl.pallas_call(
        functools.partial(_side_kernel, nt),
        grid=(2 * nt,),
        in_specs=[
            pl.BlockSpec((T, EMB), tile_map),
            pl.BlockSpec((1, 1, T), tile_map3),
            full((NCLS, EMB)), full((3 * EMB, EMB)), full((3, EMB)),
            full((EMB, EMB)), full((1, EMB)),
            full((EMB, EMB)), full((1, EMB)),
            full((EMB, 2 * EMB)), full((1, EMB)),
            full((EMB, 2 * EMB)), full((1, EMB)),
            full((1, EMB)), full((1, EMB)),
        ],
        out_specs=pl.BlockSpec((T, EMB), out_map),
        out_shape=jax.ShapeDtypeStruct((N, EMB), jnp.float32),
        scratch_shapes=[
            pltpu.VMEM((NROW, EMB), jnp.float32),
            pltpu.VMEM((NROW, 128), jnp.float32),
            pltpu.VMEM((NACC, 128), jnp.float32),
            pltpu.VMEM((NACC, EMB), jnp.float32),
            pltpu.VMEM((NCLS, EMB), jnp.float32),
        ],
    )(x, cls3, sem, Wi, bi3, Ws, row(bs), Wo, row(bo), recW, row(recb),
      gateW, row(gateb), row(ng), row(nb))
    return out


def kernel(v, c, v_sem, c_sem, params, v_class, c_class):
    p = params
    v_upd = _side(v, v_sem, p['av_Wi'], p['av_bi'], p['send_var_W'],
                  p['send_var_b'], p['av_Wo'], p['av_bo'], p['rec_var_W'],
                  p['rec_var_b'], p['gate_v_W'], p['gate_v_b'], p['norm_g'],
                  p['norm_b'], v_class, 2048)
    c_upd = _side(c, c_sem, p['ac_Wi'], p['ac_bi'], p['send_con_W'],
                  p['send_con_b'], p['ac_Wo'], p['ac_bo'], p['rec_con_W'],
                  p['rec_con_b'], p['gate_c_W'], p['gate_c_b'], p['norm_g'],
                  p['norm_b'], c_class, 2048)
    return v_upd, c_upd


# speedup vs baseline: 5.1628x; 1.1420x over previous
import functools

import jax
import jax.numpy as jnp
from jax.experimental import pallas as pl
from jax.experimental.pallas import tpu as pltpu

EMB = 256
NH = 4
DH = 64
NCLS = 8
NROW = NCLS * NH
NACC = NROW + NCLS
NEG = -1e30


def _dot(a, b, ca, cb):
    return jax.lax.dot_general(
        a, b, (((ca,), (cb,)), ((), ())), preferred_element_type=jnp.float32)


def _side_kernel(nt, x_ref, cls_ref, sem_ref, Wi_ref, bi_ref, Ws_ref, bs_ref,
                 Wo_ref, bo_ref, recW_ref, recb_ref, gateW_ref, gateb_ref,
                 ng_ref, nb_ref, out_ref,
                 qt_ref, m_ref, l_ref, Y_ref, fused_ref):
    i = pl.program_id(0)
    T = x_ref.shape[0]
    f32 = jnp.float32

    @pl.when(i == 0)
    def _init():
        sem = sem_ref[...]
        Wq = Wi_ref[0:EMB, :]
        Wk = Wi_ref[EMB:2 * EMB, :]
        bq = bi_ref[0:1, :]
        Q = _dot(sem, Wq, 1, 1) + bq
        rr = jax.lax.broadcasted_iota(jnp.int32, (NROW, NCLS), 0) // NH
        sel = (rr == jax.lax.broadcasted_iota(jnp.int32, (NROW, NCLS), 1)).astype(f32)
        Qexp = _dot(sel, Q, 1, 0)
        hh = jax.lax.broadcasted_iota(jnp.int32, (NROW, EMB), 0) % NH
        ee = jax.lax.broadcasted_iota(jnp.int32, (NROW, EMB), 1) // DH
        Qmask = Qexp * (hh == ee).astype(f32)
        t = _dot(Qmask, Wk, 1, 0)
        qt = _dot(t, Ws_ref[...], 1, 0)
        qt_ref[...] = qt * (1.0 / 8.0)
        m_ref[...] = jnp.full((NROW, 128), NEG, f32)
        l_ref[...] = jnp.zeros((NACC, 128), f32)
        Y_ref[...] = jnp.zeros((NACC, EMB), f32)

    @pl.when(i < nt)
    def _acc():
        xt = x_ref[...]
        clsrow = cls_ref[0]
        ST = _dot(qt_ref[...], xt, 1, 1)
        ccls = jax.lax.broadcasted_iota(jnp.int32, (NROW, T), 0) // NH
        msk = ccls == clsrow
        STm = jnp.where(msk, ST, NEG)
        tmax = jnp.max(STm, axis=1, keepdims=True)
        mold = m_ref[:, 0:1]
        mnew = jnp.maximum(mold, tmax)
        resc = jnp.exp(mold - mnew)
        P = jnp.where(msk, jnp.exp(STm - mnew), 0.0)
        c8 = jax.lax.broadcasted_iota(jnp.int32, (NCLS, T), 0)
        P8 = (c8 == clsrow).astype(f32)
        P40 = jnp.concatenate((P, P8), axis=0)
        resc40 = jnp.concatenate((resc, jnp.ones((NCLS, 1), f32)), axis=0)
        l_ref[...] = l_ref[...] * resc40 + jnp.sum(P40, axis=1, keepdims=True)
        Y_ref[...] = Y_ref[...] * resc40 + _dot(P40, xt, 1, 0)
        m_ref[...] = jnp.broadcast_to(mnew, (NROW, 128))

    @pl.when(i == nt - 1)
    def _fin():
        Ws = Ws_ref[...]
        bs = bs_ref[...]
        ybar = Y_ref[0:NROW, :] / l_ref[0:NROW, 0:1]
        U = _dot(ybar, Ws, 1, 1) + bs
        Wv = Wi_ref[2 * EMB:3 * EMB, :]
        bv = bi_ref[2:3, :]
        Vf = _dot(U, Wv, 1, 1) + bv
        hh2 = jax.lax.broadcasted_iota(jnp.int32, (NROW, EMB), 0) % NH
        ee2 = jax.lax.broadcasted_iota(jnp.int32, (NROW, EMB), 1) // DH
        Vm = Vf * (hh2 == ee2).astype(jnp.float32)
        rr2 = jax.lax.broadcasted_iota(jnp.int32, (NCLS, NROW), 1) // NH
        sel2 = (rr2 == jax.lax.broadcasted_iota(jnp.int32, (NCLS, NROW), 0)).astype(jnp.float32)
        attheads = _dot(sel2, Vm, 1, 0)
        att = _dot(attheads, Wo_ref[...], 1, 1) + bo_ref[...]
        old = _dot(Y_ref[NROW:NACC, :] / l_ref[NROW:NACC, 0:1], Ws, 1, 1) + bs
        sem = sem_ref[...]
        recW = recW_ref[...]
        new = (_dot(sem, recW[:, 0:EMB], 1, 1)
               + _dot(att, recW[:, EMB:2 * EMB], 1, 1) + recb_ref[...])
        gW = gateW_ref[...]
        g = jax.nn.sigmoid(_dot(old, gW[:, 0:EMB], 1, 1)
                           + _dot(new, gW[:, EMB:2 * EMB], 1, 1) + gateb_ref[...])
        fused = g * old + (1.0 - g) * new
        mu = jnp.mean(fused, axis=1, keepdims=True)
        var = jnp.mean((fused - mu) ** 2, axis=1, keepdims=True)
        fused_ref[...] = ((fused - mu) / jnp.sqrt(var + 1e-5) * ng_ref[...]
                          + nb_ref[...])

    @pl.when(i >= nt)
    def _apply():
        xt = x_ref[...]
        clsrow = cls_ref[0]
        c8 = jax.lax.broadcasted_iota(jnp.int32, (NCLS, T), 0)
        P8 = (c8 == clsrow).astype(jnp.float32)
        g = _dot(P8, fused_ref[...], 0, 0)
        out_ref[...] = g * xt


def _side(x, sem, Wi, bi, Ws, bs, Wo, bo, recW, recb, gateW, gateb, ng, nb,
          cls, T):
    N = x.shape[0]
    nt = N // T
    cls3 = cls.astype(jnp.int32).reshape(nt, 1, T)
    bi3 = bi.reshape(3, EMB)
    row = lambda a: a.reshape(1, EMB)
    full = lambda s: pl.BlockSpec(s, lambda i: (0,) * len(s))

    def tile_map(i):
        return (jnp.where(i < nt, i, i - nt), 0)

    def tile_map3(i):
        return (jnp.where(i < nt, i, i - nt), 0, 0)

    def out_map(i):
        return (jnp.maximum(i - nt, 0), 0)

    out = pl.pallas_call(
        functools.partial(_side_kernel, nt),
        grid=(2 * nt,),
        in_specs=[
            pl.BlockSpec((T, EMB), tile_map),
            pl.BlockSpec((1, 1, T), tile_map3),
            full((NCLS, EMB)), full((3 * EMB, EMB)), full((3, EMB)),
            full((EMB, EMB)), full((1, EMB)),
            full((EMB, EMB)), full((1, EMB)),
            full((EMB, 2 * EMB)), full((1, EMB)),
            full((EMB, 2 * EMB)), full((1, EMB)),
            full((1, EMB)), full((1, EMB)),
        ],
        out_specs=pl.BlockSpec((T, EMB), out_map),
        out_shape=jax.ShapeDtypeStruct((N, EMB), jnp.float32),
        scratch_shapes=[
            pltpu.VMEM((NROW, EMB), jnp.float32),
            pltpu.VMEM((NROW, 128), jnp.float32),
            pltpu.VMEM((NACC, 128), jnp.float32),
            pltpu.VMEM((NACC, EMB), jnp.float32),
            pltpu.VMEM((NCLS, EMB), jnp.float32),
        ],
    )(x, cls3, sem, Wi, bi3, Ws, row(bs), Wo, row(bo), recW, row(recb),
      gateW, row(gateb), row(ng), row(nb))
    return out


def kernel(v, c, v_sem, c_sem, params, v_class, c_class):
    p = params
    v_upd = _side(v, v_sem, p['av_Wi'], p['av_bi'], p['send_var_W'],
                  p['send_var_b'], p['av_Wo'], p['av_bo'], p['rec_var_W'],
                  p['rec_var_b'], p['gate_v_W'], p['gate_v_b'], p['norm_g'],
                  p['norm_b'], v_class, 4096)
    c_upd = _side(c, c_sem, p['ac_Wi'], p['ac_bi'], p['send_con_W'],
                  p['send_con_b'], p['ac_Wo'], p['ac_bo'], p['rec_con_W'],
                  p['rec_con_b'], p['gate_c_W'], p['gate_c_b'], p['norm_g'],
                  p['norm_b'], c_class, 4096)
    return v_upd, c_upd


# speedup vs baseline: 5.4628x; 1.0581x over previous
import functools

import jax
import jax.numpy as jnp
from jax.experimental import pallas as pl
from jax.experimental.pallas import tpu as pltpu

EMB = 256
NH = 4
DH = 64
NCLS = 8
NROW = NCLS * NH
NACC = NROW + NCLS
NEG = -1e30


def _dot(a, b, ca, cb):
    return jax.lax.dot_general(
        a, b, (((ca,), (cb,)), ((), ())), preferred_element_type=jnp.float32)


def _side_kernel(nt, x_ref, cls_ref, sem_ref, Wi_ref, bi_ref, Ws_ref, bs_ref,
                 Wo_ref, bo_ref, recW_ref, recb_ref, gateW_ref, gateb_ref,
                 ng_ref, nb_ref, out_ref,
                 qt_ref, m_ref, l_ref, Y_ref, fused_ref):
    i = pl.program_id(0)
    T = x_ref.shape[0]
    f32 = jnp.float32

    @pl.when(i == 0)
    def _init():
        sem = sem_ref[...]
        Wq = Wi_ref[0:EMB, :]
        Wk = Wi_ref[EMB:2 * EMB, :]
        bq = bi_ref[0:1, :]
        Q = _dot(sem, Wq, 1, 1) + bq
        rr = jax.lax.broadcasted_iota(jnp.int32, (NROW, NCLS), 0) // NH
        sel = (rr == jax.lax.broadcasted_iota(jnp.int32, (NROW, NCLS), 1)).astype(f32)
        Qexp = _dot(sel, Q, 1, 0)
        hh = jax.lax.broadcasted_iota(jnp.int32, (NROW, EMB), 0) % NH
        ee = jax.lax.broadcasted_iota(jnp.int32, (NROW, EMB), 1) // DH
        Qmask = Qexp * (hh == ee).astype(f32)
        t = _dot(Qmask, Wk, 1, 0)
        qt = _dot(t, Ws_ref[...], 1, 0)
        qt_ref[...] = qt * (1.0 / 8.0)
        m_ref[...] = jnp.full((NROW, 128), NEG, f32)
        l_ref[...] = jnp.zeros((NACC, 128), f32)
        Y_ref[...] = jnp.zeros((NACC, EMB), f32)

    @pl.when(i < nt)
    def _acc():
        xt = x_ref[...]
        clsrow = cls_ref[0]
        ST = _dot(qt_ref[...], xt, 1, 1)
        ccls = jax.lax.broadcasted_iota(jnp.int32, (NROW, T), 0) // NH
        msk = ccls == clsrow
        STm = jnp.where(msk, ST, NEG)
        tmax = jnp.max(STm, axis=1, keepdims=True)
        mold = m_ref[:, 0:1]
        mnew = jnp.maximum(mold, tmax)
        resc = jnp.exp(mold - mnew)
        P = jnp.where(msk, jnp.exp(STm - mnew), 0.0)
        c8 = jax.lax.broadcasted_iota(jnp.int32, (NCLS, T), 0)
        P8 = (c8 == clsrow).astype(f32)
        P40 = jnp.concatenate((P, P8), axis=0)
        resc40 = jnp.concatenate((resc, jnp.ones((NCLS, 1), f32)), axis=0)
        l_ref[...] = l_ref[...] * resc40 + jnp.sum(P40, axis=1, keepdims=True)
        Y_ref[...] = Y_ref[...] * resc40 + _dot(P40, xt, 1, 0)
        m_ref[...] = jnp.broadcast_to(mnew, (NROW, 128))

    @pl.when(i == nt - 1)
    def _fin():
        Ws = Ws_ref[...]
        bs = bs_ref[...]
        ybar = Y_ref[0:NROW, :] / l_ref[0:NROW, 0:1]
        U = _dot(ybar, Ws, 1, 1) + bs
        Wv = Wi_ref[2 * EMB:3 * EMB, :]
        bv = bi_ref[2:3, :]
        Vf = _dot(U, Wv, 1, 1) + bv
        hh2 = jax.lax.broadcasted_iota(jnp.int32, (NROW, EMB), 0) % NH
        ee2 = jax.lax.broadcasted_iota(jnp.int32, (NROW, EMB), 1) // DH
        Vm = Vf * (hh2 == ee2).astype(jnp.float32)
        rr2 = jax.lax.broadcasted_iota(jnp.int32, (NCLS, NROW), 1) // NH
        sel2 = (rr2 == jax.lax.broadcasted_iota(jnp.int32, (NCLS, NROW), 0)).astype(jnp.float32)
        attheads = _dot(sel2, Vm, 1, 0)
        att = _dot(attheads, Wo_ref[...], 1, 1) + bo_ref[...]
        old = _dot(Y_ref[NROW:NACC, :] / l_ref[NROW:NACC, 0:1], Ws, 1, 1) + bs
        sem = sem_ref[...]
        recW = recW_ref[...]
        new = (_dot(sem, recW[:, 0:EMB], 1, 1)
               + _dot(att, recW[:, EMB:2 * EMB], 1, 1) + recb_ref[...])
        gW = gateW_ref[...]
        g = jax.nn.sigmoid(_dot(old, gW[:, 0:EMB], 1, 1)
                           + _dot(new, gW[:, EMB:2 * EMB], 1, 1) + gateb_ref[...])
        fused = g * old + (1.0 - g) * new
        mu = jnp.mean(fused, axis=1, keepdims=True)
        var = jnp.mean((fused - mu) ** 2, axis=1, keepdims=True)
        fused_ref[...] = ((fused - mu) / jnp.sqrt(var + 1e-5) * ng_ref[...]
                          + nb_ref[...])

    @pl.when(i >= nt)
    def _apply():
        xt = x_ref[...]
        clsrow = cls_ref[0]
        c8 = jax.lax.broadcasted_iota(jnp.int32, (NCLS, T), 0)
        P8 = (c8 == clsrow).astype(jnp.float32)
        g = _dot(P8, fused_ref[...], 0, 0)
        out_ref[...] = g * xt


def _side(x, sem, Wi, bi, Ws, bs, Wo, bo, recW, recb, gateW, gateb, ng, nb,
          cls, T):
    N = x.shape[0]
    nt = N // T
    cls3 = cls.astype(jnp.int32).reshape(nt, 1, T)
    bi3 = bi.reshape(3, EMB)
    row = lambda a: a.reshape(1, EMB)
    full = lambda s: pl.BlockSpec(s, lambda i: (0,) * len(s))

    def tile_map(i):
        return (jnp.where(i < nt, i, i - nt), 0)

    def tile_map3(i):
        return (jnp.where(i < nt, i, i - nt), 0, 0)

    def out_map(i):
        return (jnp.maximum(i - nt, 0), 0)

    out = pl.pallas_call(
        functools.partial(_side_kernel, nt),
        grid=(2 * nt,),
        in_specs=[
            pl.BlockSpec((T, EMB), tile_map),
            pl.BlockSpec((1, 1, T), tile_map3),
            full((NCLS, EMB)), full((3 * EMB, EMB)), full((3, EMB)),
            full((EMB, EMB)), full((1, EMB)),
            full((EMB, EMB)), full((1, EMB)),
            full((EMB, 2 * EMB)), full((1, EMB)),
            full((EMB, 2 * EMB)), full((1, EMB)),
            full((1, EMB)), full((1, EMB)),
        ],
        out_specs=pl.BlockSpec((T, EMB), out_map),
        out_shape=jax.ShapeDtypeStruct((N, EMB), jnp.float32),
        scratch_shapes=[
            pltpu.VMEM((NROW, EMB), jnp.float32),
            pltpu.VMEM((NROW, 128), jnp.float32),
            pltpu.VMEM((NACC, 128), jnp.float32),
            pltpu.VMEM((NACC, EMB), jnp.float32),
            pltpu.VMEM((NCLS, EMB), jnp.float32),
        ],
    )(x, cls3, sem, Wi, bi3, Ws, row(bs), Wo, row(bo), recW, row(recb),
      gateW, row(gateb), row(ng), row(nb))
    return out


def kernel(v, c, v_sem, c_sem, params, v_class, c_class):
    p = params
    v_upd = _side(v, v_sem, p['av_Wi'], p['av_bi'], p['send_var_W'],
                  p['send_var_b'], p['av_Wo'], p['av_bo'], p['rec_var_W'],
                  p['rec_var_b'], p['gate_v_W'], p['gate_v_b'], p['norm_g'],
                  p['norm_b'], v_class, 8192)
    c_upd = _side(c, c_sem, p['ac_Wi'], p['ac_bi'], p['send_con_W'],
                  p['send_con_b'], p['ac_Wo'], p['ac_bo'], p['rec_con_W'],
                  p['rec_con_b'], p['gate_c_W'], p['gate_c_b'], p['norm_g'],
                  p['norm_b'], c_class, 8192)
    return v_upd, c_upd


# speedup vs baseline: 5.8575x; 1.0722x over previous
import functools

import jax
import jax.numpy as jnp
from jax.experimental import pallas as pl
from jax.experimental.pallas import tpu as pltpu

EMB = 256
NH = 4
DH = 64
NCLS = 8
NROW = NCLS * NH
NACC = NROW + NCLS
NEG = -1e30


def _dot(a, b, ca, cb):
    return jax.lax.dot_general(
        a, b, (((ca,), (cb,)), ((), ())), preferred_element_type=jnp.float32)


def _side_kernel(nt, x_ref, cls_ref, sem_ref, Wi_ref, bi_ref, Ws_ref, bs_ref,
                 Wo_ref, bo_ref, recW_ref, recb_ref, gateW_ref, gateb_ref,
                 ng_ref, nb_ref, out_ref,
                 qt_ref, m_ref, l_ref, Y_ref, fused_ref, xsave_ref):
    i = pl.program_id(0)
    T = x_ref.shape[0]
    f32 = jnp.float32

    @pl.when(i == 0)
    def _init():
        sem = sem_ref[...]
        Wq = Wi_ref[0:EMB, :]
        Wk = Wi_ref[EMB:2 * EMB, :]
        bq = bi_ref[0:1, :]
        Q = _dot(sem, Wq, 1, 1) + bq
        rr = jax.lax.broadcasted_iota(jnp.int32, (NROW, NCLS), 0) // NH
        sel = (rr == jax.lax.broadcasted_iota(jnp.int32, (NROW, NCLS), 1)).astype(f32)
        Qexp = _dot(sel, Q, 1, 0)
        hh = jax.lax.broadcasted_iota(jnp.int32, (NROW, EMB), 0) % NH
        ee = jax.lax.broadcasted_iota(jnp.int32, (NROW, EMB), 1) // DH
        Qmask = Qexp * (hh == ee).astype(f32)
        t = _dot(Qmask, Wk, 1, 0)
        qt = _dot(t, Ws_ref[...], 1, 0)
        qt_ref[...] = qt * (1.0 / 8.0)
        m_ref[...] = jnp.full((NROW, 128), NEG, f32)
        l_ref[...] = jnp.zeros((NACC, 128), f32)
        Y_ref[...] = jnp.zeros((NACC, EMB), f32)

    @pl.when(i < nt)
    def _acc():
        xt = x_ref[...]
        xsave_ref[pl.ds(i * T, T), :] = xt
        clsrow = cls_ref[i]
        ST = _dot(qt_ref[...], xt, 1, 1)
        ccls = jax.lax.broadcasted_iota(jnp.int32, (NROW, T), 0) // NH
        msk = ccls == clsrow
        STm = jnp.where(msk, ST, NEG)
        tmax = jnp.max(STm, axis=1, keepdims=True)
        mold = m_ref[:, 0:1]
        mnew = jnp.maximum(mold, tmax)
        resc = jnp.exp(mold - mnew)
        P = jnp.where(msk, jnp.exp(STm - mnew), 0.0)
        c8 = jax.lax.broadcasted_iota(jnp.int32, (NCLS, T), 0)
        P8 = (c8 == clsrow).astype(f32)
        P40 = jnp.concatenate((P, P8), axis=0)
        resc40 = jnp.concatenate((resc, jnp.ones((NCLS, 1), f32)), axis=0)
        l_ref[...] = l_ref[...] * resc40 + jnp.sum(P40, axis=1, keepdims=True)
        Y_ref[...] = Y_ref[...] * resc40 + _dot(P40, xt, 1, 0)
        m_ref[...] = jnp.broadcast_to(mnew, (NROW, 128))

    @pl.when(i == nt - 1)
    def _fin():
        Ws = Ws_ref[...]
        bs = bs_ref[...]
        ybar = Y_ref[0:NROW, :] / l_ref[0:NROW, 0:1]
        U = _dot(ybar, Ws, 1, 1) + bs
        Wv = Wi_ref[2 * EMB:3 * EMB, :]
        bv = bi_ref[2:3, :]
        Vf = _dot(U, Wv, 1, 1) + bv
        hh2 = jax.lax.broadcasted_iota(jnp.int32, (NROW, EMB), 0) % NH
        ee2 = jax.lax.broadcasted_iota(jnp.int32, (NROW, EMB), 1) // DH
        Vm = Vf * (hh2 == ee2).astype(jnp.float32)
        rr2 = jax.lax.broadcasted_iota(jnp.int32, (NCLS, NROW), 1) // NH
        sel2 = (rr2 == jax.lax.broadcasted_iota(jnp.int32, (NCLS, NROW), 0)).astype(jnp.float32)
        attheads = _dot(sel2, Vm, 1, 0)
        att = _dot(attheads, Wo_ref[...], 1, 1) + bo_ref[...]
        old = _dot(Y_ref[NROW:NACC, :] / l_ref[NROW:NACC, 0:1], Ws, 1, 1) + bs
        sem = sem_ref[...]
        recW = recW_ref[...]
        new = (_dot(sem, recW[:, 0:EMB], 1, 1)
               + _dot(att, recW[:, EMB:2 * EMB], 1, 1) + recb_ref[...])
        gW = gateW_ref[...]
        g = jax.nn.sigmoid(_dot(old, gW[:, 0:EMB], 1, 1)
                           + _dot(new, gW[:, EMB:2 * EMB], 1, 1) + gateb_ref[...])
        fused = g * old + (1.0 - g) * new
        mu = jnp.mean(fused, axis=1, keepdims=True)
        var = jnp.mean((fused - mu) ** 2, axis=1, keepdims=True)
        fused_ref[...] = ((fused - mu) / jnp.sqrt(var + 1e-5) * ng_ref[...]
                          + nb_ref[...])

    @pl.when(i >= nt)
    def _apply():
        j = i - nt
        xt = xsave_ref[pl.ds(j * T, T), :]
        clsrow = cls_ref[j]
        c8 = jax.lax.broadcasted_iota(jnp.int32, (NCLS, T), 0)
        P8 = (c8 == clsrow).astype(jnp.float32)
        g = _dot(P8, fused_ref[...], 0, 0)
        out_ref[...] = g * xt


def _side(x, sem, Wi, bi, Ws, bs, Wo, bo, recW, recb, gateW, gateb, ng, nb,
          cls, T):
    N = x.shape[0]
    nt = N // T
    cls3 = cls.astype(jnp.int32).reshape(nt, 1, T)
    bi3 = bi.reshape(3, EMB)
    row = lambda a: a.reshape(1, EMB)
    full = lambda s: pl.BlockSpec(s, lambda i: (0,) * len(s))

    def tile_map(i):
        return (jnp.minimum(i, nt - 1), 0)

    def out_map(i):
        return (jnp.maximum(i - nt, 0), 0)

    out = pl.pallas_call(
        functools.partial(_side_kernel, nt),
        grid=(2 * nt,),
        in_specs=[
            pl.BlockSpec((T, EMB), tile_map),
            full((nt, 1, T)),
            full((NCLS, EMB)), full((3 * EMB, EMB)), full((3, EMB)),
            full((EMB, EMB)), full((1, EMB)),
            full((EMB, EMB)), full((1, EMB)),
            full((EMB, 2 * EMB)), full((1, EMB)),
            full((EMB, 2 * EMB)), full((1, EMB)),
            full((1, EMB)), full((1, EMB)),
        ],
        out_specs=pl.BlockSpec((T, EMB), out_map),
        out_shape=jax.ShapeDtypeStruct((N, EMB), jnp.float32),
        scratch_shapes=[
            pltpu.VMEM((NROW, EMB), jnp.float32),
            pltpu.VMEM((NROW, 128), jnp.float32),
            pltpu.VMEM((NACC, 128), jnp.float32),
            pltpu.VMEM((NACC, EMB), jnp.float32),
            pltpu.VMEM((NCLS, EMB), jnp.float32),
            pltpu.VMEM((N, EMB), jnp.float32),
        ],
    )(x, cls3, sem, Wi, bi3, Ws, row(bs), Wo, row(bo), recW, row(recb),
      gateW, row(gateb), row(ng), row(nb))
    return out


def kernel(v, c, v_sem, c_sem, params, v_class, c_class):
    p = params
    v_upd = _side(v, v_sem, p['av_Wi'], p['av_bi'], p['send_var_W'],
                  p['send_var_b'], p['av_Wo'], p['av_bo'], p['rec_var_W'],
                  p['rec_var_b'], p['gate_v_W'], p['gate_v_b'], p['norm_g'],
                  p['norm_b'], v_class, 8192)
    c_upd = _side(c, c_sem, p['ac_Wi'], p['ac_bi'], p['send_con_W'],
                  p['send_con_b'], p['ac_Wo'], p['ac_bo'], p['rec_con_W'],
                  p['rec_con_b'], p['gate_c_W'], p['gate_c_b'], p['norm_g'],
                  p['norm_b'], c_class, 8192)
    return v_upd, c_upd


# speedup vs baseline: 6.0777x; 1.0376x over previous
import functools

import jax
import jax.numpy as jnp
from jax.experimental import pallas as pl
from jax.experimental.pallas import tpu as pltpu

EMB = 256
NH = 4
DH = 64
NCLS = 8
NROW = NCLS * NH
NACC = NROW + NCLS
NEG = -1e30


def _dot(a, b, ca, cb):
    return jax.lax.dot_general(
        a, b, (((ca,), (cb,)), ((), ())), preferred_element_type=jnp.float32)


def _side_kernel(nt, x_ref, cls_ref, sem_ref, Wi_ref, bi_ref, Ws_ref, bs_ref,
                 Wo_ref, bo_ref, recW_ref, recb_ref, gateW_ref, gateb_ref,
                 ng_ref, nb_ref, out_ref,
                 qt_ref, m_ref, l_ref, Y_ref, fused_ref, xsave_ref):
    i = pl.program_id(0)
    T = x_ref.shape[0]
    f32 = jnp.float32

    @pl.when(i == 0)
    def _init():
        sem = sem_ref[...]
        Wq = Wi_ref[0:EMB, :]
        Wk = Wi_ref[EMB:2 * EMB, :]
        bq = bi_ref[0:1, :]
        Q = _dot(sem, Wq, 1, 1) + bq
        rr = jax.lax.broadcasted_iota(jnp.int32, (NROW, NCLS), 0) // NH
        sel = (rr == jax.lax.broadcasted_iota(jnp.int32, (NROW, NCLS), 1)).astype(f32)
        Qexp = _dot(sel, Q, 1, 0)
        hh = jax.lax.broadcasted_iota(jnp.int32, (NROW, EMB), 0) % NH
        ee = jax.lax.broadcasted_iota(jnp.int32, (NROW, EMB), 1) // DH
        Qmask = Qexp * (hh == ee).astype(f32)
        t = _dot(Qmask, Wk, 1, 0)
        qt = _dot(t, Ws_ref[...], 1, 0)
        qt_ref[...] = qt * (1.0 / 8.0)
        m_ref[...] = jnp.full((NROW, 128), NEG, f32)
        l_ref[...] = jnp.zeros((NACC, 128), f32)
        Y_ref[...] = jnp.zeros((NACC, EMB), f32)

    @pl.when(i < nt)
    def _acc():
        xt = x_ref[...]
        xsave_ref[pl.ds(i * T, T), :] = xt
        clsrow = cls_ref[i]
        ST = _dot(qt_ref[...], xt, 1, 1)
        ccls = jax.lax.broadcasted_iota(jnp.int32, (NROW, T), 0) // NH
        msk = ccls == clsrow
        STm = jnp.where(msk, ST, NEG)
        tmax = jnp.max(STm, axis=1, keepdims=True)
        mold = m_ref[:, 0:1]
        mnew = jnp.maximum(mold, tmax)
        resc = jnp.exp(mold - mnew)
        P = jnp.where(msk, jnp.exp(STm - mnew), 0.0)
        c8 = jax.lax.broadcasted_iota(jnp.int32, (NCLS, T), 0)
        P8 = (c8 == clsrow).astype(f32)
        P40 = jnp.concatenate((P, P8), axis=0)
        resc40 = jnp.concatenate((resc, jnp.ones((NCLS, 1), f32)), axis=0)
        l_ref[...] = l_ref[...] * resc40 + jnp.sum(P40, axis=1, keepdims=True)
        Y_ref[...] = Y_ref[...] * resc40 + _dot(P40, xt, 1, 0)
        m_ref[...] = jnp.broadcast_to(mnew, (NROW, 128))

    @pl.when(i == nt - 1)
    def _fin():
        Ws = Ws_ref[...]
        bs = bs_ref[...]
        ybar = Y_ref[0:NROW, :] / l_ref[0:NROW, 0:1]
        U = _dot(ybar, Ws, 1, 1) + bs
        Wv = Wi_ref[2 * EMB:3 * EMB, :]
        bv = bi_ref[2:3, :]
        Vf = _dot(U, Wv, 1, 1) + bv
        hh2 = jax.lax.broadcasted_iota(jnp.int32, (NROW, EMB), 0) % NH
        ee2 = jax.lax.broadcasted_iota(jnp.int32, (NROW, EMB), 1) // DH
        Vm = Vf * (hh2 == ee2).astype(jnp.float32)
        rr2 = jax.lax.broadcasted_iota(jnp.int32, (NCLS, NROW), 1) // NH
        sel2 = (rr2 == jax.lax.broadcasted_iota(jnp.int32, (NCLS, NROW), 0)).astype(jnp.float32)
        attheads = _dot(sel2, Vm, 1, 0)
        att = _dot(attheads, Wo_ref[...], 1, 1) + bo_ref[...]
        old = _dot(Y_ref[NROW:NACC, :] / l_ref[NROW:NACC, 0:1], Ws, 1, 1) + bs
        sem = sem_ref[...]
        recW = recW_ref[...]
        new = (_dot(sem, recW[:, 0:EMB], 1, 1)
               + _dot(att, recW[:, EMB:2 * EMB], 1, 1) + recb_ref[...])
        gW = gateW_ref[...]
        g = jax.nn.sigmoid(_dot(old, gW[:, 0:EMB], 1, 1)
                           + _dot(new, gW[:, EMB:2 * EMB], 1, 1) + gateb_ref[...])
        fused = g * old + (1.0 - g) * new
        mu = jnp.mean(fused, axis=1, keepdims=True)
        var = jnp.mean((fused - mu) ** 2, axis=1, keepdims=True)
        fused_ref[...] = ((fused - mu) / jnp.sqrt(var + 1e-5) * ng_ref[...]
                          + nb_ref[...])

    @pl.when(i >= nt)
    def _apply():
        j = i - nt
        xt = xsave_ref[pl.ds(j * T, T), :]
        clsrow = cls_ref[j]
        c8 = jax.lax.broadcasted_iota(jnp.int32, (NCLS, T), 0)
        P8 = (c8 == clsrow).astype(jnp.float32)
        g = _dot(P8, fused_ref[...], 0, 0)
        out_ref[...] = g * xt


def _side(x, sem, Wi, bi, Ws, bs, Wo, bo, recW, recb, gateW, gateb, ng, nb,
          cls, T):
    N = x.shape[0]
    nt = N // T
    cls3 = cls.astype(jnp.int32).reshape(nt, 1, T)
    bi3 = bi.reshape(3, EMB)
    row = lambda a: a.reshape(1, EMB)
    full = lambda s: pl.BlockSpec(s, lambda i: (0,) * len(s))

    def tile_map(i):
        return (jnp.minimum(i, nt - 1), 0)

    def out_map(i):
        return (jnp.maximum(i - nt, 0), 0)

    out = pl.pallas_call(
        functools.partial(_side_kernel, nt),
        grid=(2 * nt,),
        in_specs=[
            pl.BlockSpec((T, EMB), tile_map),
            full((nt, 1, T)),
            full((NCLS, EMB)), full((3 * EMB, EMB)), full((3, EMB)),
            full((EMB, EMB)), full((1, EMB)),
            full((EMB, EMB)), full((1, EMB)),
            full((EMB, 2 * EMB)), full((1, EMB)),
            full((EMB, 2 * EMB)), full((1, EMB)),
            full((1, EMB)), full((1, EMB)),
        ],
        out_specs=pl.BlockSpec((T, EMB), out_map),
        out_shape=jax.ShapeDtypeStruct((N, EMB), jnp.float32),
        scratch_shapes=[
            pltpu.VMEM((NROW, EMB), jnp.float32),
            pltpu.VMEM((NROW, 128), jnp.float32),
            pltpu.VMEM((NACC, 128), jnp.float32),
            pltpu.VMEM((NACC, EMB), jnp.float32),
            pltpu.VMEM((NCLS, EMB), jnp.float32),
            pltpu.VMEM((N, EMB), jnp.float32),
        ],
    )(x, cls3, sem, Wi, bi3, Ws, row(bs), Wo, row(bo), recW, row(recb),
      gateW, row(gateb), row(ng), row(nb))
    return out


def kernel(v, c, v_sem, c_sem, params, v_class, c_class):
    p = params
    v_upd = _side(v, v_sem, p['av_Wi'], p['av_bi'], p['send_var_W'],
                  p['send_var_b'], p['av_Wo'], p['av_bo'], p['rec_var_W'],
                  p['rec_var_b'], p['gate_v_W'], p['gate_v_b'], p['norm_g'],
                  p['norm_b'], v_class, 4096)
    c_upd = _side(c, c_sem, p['ac_Wi'], p['ac_bi'], p['send_con_W'],
                  p['send_con_b'], p['ac_Wo'], p['ac_bo'], p['rec_con_W'],
                  p['rec_con_b'], p['gate_c_W'], p['gate_c_b'], p['norm_g'],
                  p['norm_b'], c_class, 4096)
    return v_upd, c_upd
